# baseline (speedup 1.0000x reference)
import functools

import jax
import jax.numpy as jnp
from jax import lax
from jax.experimental import pallas as pl
from jax.experimental.pallas import tpu as pltpu
from jax.experimental.pallas import tpu_sc as plsc

N = 10000
NP = 10240
G = 64
D_IN = 200
H = 16

EC = 128
NTILES = 32
ER = 2816
RT = ER // NTILES
RTD = ER // 16

_MAGIC = 0x5F3759DF


def _invsqrt16(d):
    i = plsc.bitcast(d, jnp.int32)
    y = plsc.bitcast(_MAGIC - lax.shift_right_arithmetic(i, 1), jnp.float32)
    for _ in range(3):
        y = y * (1.5 - 0.5 * d * y * y)
    return y


NB = 4
ROWB = EC * H * 4
DEGQ = 16


def _sc_layer_body(compute_dinv, *refs):
    if compute_dinv:
        (src_hbm, dst_hbm, ew_hbm, feat_hbm, z_nodes, z_deg,
         part_out, dinv_out,
         src_blk, dst_blk, ew_blk, dinv_v, dtmp,
         r0, r1, r2, r3, cvals,
         acc_sh, feat_sh, deg_sh,
         g0, g1, g2, g3, s0, s1, s2, s3, dsem) = refs
    else:
        (src_hbm, dst_hbm, ew_hbm, feat_hbm, z_nodes, dinv_hbm,
         part_out,
         src_blk, dst_blk, ew_blk, dinv_v,
         r0, r1, r2, r3, cvals,
         acc_sh, feat_sh,
         g0, g1, g2, g3, s0, s1, s2, s3) = refs
    rows = (r0, r1, r2, r3)
    gsem = (g0, g1, g2, g3)
    ssem = (s0, s1, s2, s3)

    cid = lax.axis_index("c")
    sid = lax.axis_index("s")
    wid = cid * 16 + sid

    @pl.when(sid == 0)
    def _():
        pltpu.sync_copy(z_nodes, acc_sh)
        if compute_dinv:
            pltpu.sync_copy(z_deg, deg_sh)

    @pl.when(sid == 1)
    def _():
        pltpu.sync_copy(feat_hbm, feat_sh)

    plsc.subcore_barrier()

    if compute_dinv:
        pltpu.sync_copy(dst_hbm.at[pl.ds(sid * RTD, RTD)], dst_blk)
        pltpu.sync_copy(ew_hbm.at[pl.ds(sid * RTD, RTD)], ew_blk)

        def deg_step(j, _):
            pltpu.async_copy(ew_blk.at[j], deg_sh.at[dst_blk.at[j]], dsem,
                             add=True)

            @pl.when(j >= DEGQ)
            def _():
                pltpu.make_async_copy(
                    ew_blk.at[j], deg_sh.at[dst_blk.at[j]], dsem).wait()

            return 0

        lax.fori_loop(0, RTD, deg_step, 0)
        for _ in range(DEGQ):
            pltpu.make_async_copy(
                ew_blk.at[0], deg_sh.at[dst_blk.at[0]], dsem).wait()
        plsc.subcore_barrier()

        pltpu.sync_copy(deg_sh.at[pl.ds(sid * 640, 640)], dtmp)

        def inv_step(k, _):
            dtmp[pl.ds(k * 16, 16)] = _invsqrt16(dtmp[pl.ds(k * 16, 16)])
            return 0

        lax.fori_loop(0, 40, inv_step, 0)
        pltpu.sync_copy(dtmp, deg_sh.at[pl.ds(sid * 640, 640)])
        plsc.subcore_barrier()

        @pl.when(jnp.logical_and(cid == 0, sid == 0))
        def _():
            pltpu.sync_copy(deg_sh, dinv_out)

        pltpu.sync_copy(deg_sh, dinv_v)
    else:
        pltpu.sync_copy(dinv_hbm, dinv_v)

    pltpu.sync_copy(src_hbm.at[pl.ds(wid * RT, RT)], src_blk)
    if compute_dinv:
        pltpu.sync_copy(dst_hbm.at[pl.ds(wid * RT, RT)], dst_blk.at[pl.ds(0, RT)])
        pltpu.sync_copy(ew_hbm.at[pl.ds(wid * RT, RT)], ew_blk.at[pl.ds(0, RT)])
    else:
        pltpu.sync_copy(dst_hbm.at[pl.ds(wid * RT, RT)], dst_blk)
        pltpu.sync_copy(ew_hbm.at[pl.ds(wid * RT, RT)], ew_blk)

    pltpu.async_copy(feat_sh.at[src_blk.at[0]], rows[0], gsem[0])

    def msg_outer(jo, _):
        for b in range(NB):
            j = jo + b
            nb = (b + 1) % NB

            @pl.when(j + 1 < RT)
            def _():
                @pl.when(j + 1 >= NB)
                def _():
                    pltpu.make_async_copy(
                        rows[nb], acc_sh.at[dst_blk.at[j]], ssem[nb]).wait()

                pltpu.async_copy(feat_sh.at[src_blk.at[j + 1]], rows[nb],
                                 gsem[nb])

            for g in range(EC // 16):
                sl = pl.ds(g * 16, 16)
                c = (ew_blk[j, sl]
                     * plsc.load_gather(dinv_v, [src_blk[j, sl]])
                     * plsc.load_gather(dinv_v, [dst_blk[j, sl]]))
                cvals[sl] = c

            pltpu.make_async_copy(
                feat_sh.at[src_blk.at[j]], rows[b], gsem[b]).wait()

            for e in range(EC):
                c_b = plsc.load_gather(cvals, [jnp.full((16,), e, jnp.int32)])
                rows[b][e, :] = rows[b][e, :] * c_b

            pltpu.async_copy(rows[b], acc_sh.at[dst_blk.at[j]], ssem[b],
                             add=True)
        return 0

    lax.fori_loop(0, RT // NB, lambda i, c: msg_outer(i * NB, c), 0)
    for b in range(NB):
        pltpu.make_async_copy(
            rows[b], acc_sh.at[dst_blk.at[0]], ssem[b]).wait()
    plsc.subcore_barrier()

    @pl.when(sid == 0)
    def _():
        pltpu.sync_copy(acc_sh, part_out.at[cid])


@functools.cache
def _make_sc_layer(compute_dinv):
    out_type = [jax.ShapeDtypeStruct((2, N, H), jnp.float32)]
    if compute_dinv:
        out_type.append(jax.ShapeDtypeStruct((NP,), jnp.float32))
    scratch = [
        pltpu.VMEM((RT, EC), jnp.int32),
        pltpu.VMEM((RTD if compute_dinv else RT, EC), jnp.int32),
        pltpu.VMEM((RTD if compute_dinv else RT, EC), jnp.float32),
        pltpu.VMEM((NP,), jnp.float32),
    ]
    if compute_dinv:
        scratch.append(pltpu.VMEM((640,), jnp.float32))
    scratch += [pltpu.VMEM((EC, H), jnp.float32) for _ in range(NB)]
    scratch += [
        pltpu.VMEM((EC,), jnp.float32),
        pltpu.VMEM_SHARED((N, H), jnp.float32),
        pltpu.VMEM_SHARED((N, H), jnp.float32),
    ]
    if compute_dinv:
        scratch.append(pltpu.VMEM_SHARED((NP,), jnp.float32))
    scratch += [pltpu.SemaphoreType.DMA] * (2 * NB)
    if compute_dinv:
        scratch.append(pltpu.SemaphoreType.DMA)

    return pl.kernel(
        functools.partial(_sc_layer_body, compute_dinv),
        out_type=tuple(out_type),
        mesh=plsc.VectorSubcoreMesh(core_axis_name="c", subcore_axis_name="s"),
        scratch_types=tuple(scratch),
        compiler_params=pltpu.CompilerParams(
            needs_layout_passes=False, use_tc_tiling_on_sc=False),
        name="gcn_sc_layer1" if compute_dinv else "gcn_sc_layer2",
    )


def _mm1_body(x_ref, w_ref, o_ref):
    o_ref[...] = jnp.dot(x_ref[...], w_ref[...],
                         preferred_element_type=jnp.float32)


def _mm2_body(part_ref, b_ref, w_ref, o_ref):
    h = jnp.maximum(part_ref[0] + part_ref[1] + b_ref[...], 0.0)
    o_ref[...] = jnp.dot(h, w_ref[...], preferred_element_type=jnp.float32)


def _pool_body(part_ref, b_ref, batch_ref, wl_ref, bl_ref, o_ref):
    h = jnp.maximum(part_ref[0] + part_ref[1] + b_ref[...], 0.0)
    iot = lax.broadcasted_iota(jnp.int32, (G, N), 0)
    oh = (iot == batch_ref[...]).astype(jnp.float32)
    sums = jnp.dot(oh, h, preferred_element_type=jnp.float32)
    cnts = jnp.sum(oh, axis=1, keepdims=True)
    g = sums / jnp.maximum(cnts, 1.0)
    o_ref[...] = jnp.dot(g, wl_ref[...],
                         preferred_element_type=jnp.float32) + bl_ref[...]


def kernel(x, edge_index, edge_attr, batch, W1, b1, W2, b2, Wl, bl):
    E = edge_attr.shape[0]
    pad = ER * EC - E - N
    loop_idx = jnp.arange(N, dtype=jnp.int32)
    zpad_i = jnp.zeros((pad,), jnp.int32)
    src_f = jnp.concatenate([edge_index[0], loop_idx, zpad_i]).reshape(ER, EC)
    dst_f = jnp.concatenate([edge_index[1], loop_idx, zpad_i]).reshape(ER, EC)
    ew_f = jnp.concatenate(
        [edge_attr, jnp.ones((N,), jnp.float32), jnp.zeros((pad,), jnp.float32)]
    ).reshape(ER, EC)

    z_nodes = jnp.zeros((N, H), jnp.float32)
    z_deg = jnp.zeros((NP,), jnp.float32)

    W2p = jnp.pad(W2, ((0, 0), (0, H - W2.shape[1])))
    b2p = jnp.pad(b2, (0, H - b2.shape[0])).reshape(1, H)
    Wlp = jnp.pad(Wl, ((0, H - Wl.shape[0]), (0, 0)))
    b1r = b1.reshape(1, H)

    xw1 = pl.pallas_call(
        _mm1_body,
        grid=(10,),
        in_specs=[
            pl.BlockSpec((N // 10, D_IN), lambda i: (i, 0)),
            pl.BlockSpec((D_IN, H), lambda i: (0, 0)),
        ],
        out_specs=pl.BlockSpec((N // 10, H), lambda i: (i, 0)),
        out_shape=jax.ShapeDtypeStruct((N, H), jnp.float32),
        name="gcn_mm1",
    )(x, W1)

    part1, dinv = _make_sc_layer(True)(src_f, dst_f, ew_f, xw1, z_nodes, z_deg)

    xw2 = pl.pallas_call(
        _mm2_body,
        out_shape=jax.ShapeDtypeStruct((N, H), jnp.float32),
        name="gcn_mm2",
    )(part1, b1r, W2p)

    (part2,) = _make_sc_layer(False)(src_f, dst_f, ew_f, xw2, z_nodes, dinv)

    out = pl.pallas_call(
        _pool_body,
        out_shape=jax.ShapeDtypeStruct((G, 1), jnp.float32),
        name="gcn_pool",
    )(part2, b2p, batch.reshape(1, N), Wlp, bl.reshape(1, 1))

    return out.reshape(-1)


# speedup vs baseline: 1.0980x; 1.0980x over previous
import functools

import jax
import jax.numpy as jnp
from jax import lax
from jax.experimental import pallas as pl
from jax.experimental.pallas import tpu as pltpu
from jax.experimental.pallas import tpu_sc as plsc

N = 10000
NP = 10240
G = 64
D_IN = 200
H = 16

EC = 128
NTILES = 32
ER = 2816
RT = ER // NTILES
RTD = ER // 16

_MAGIC = 0x5F3759DF


def _invsqrt16(d):
    i = plsc.bitcast(d, jnp.int32)
    y = plsc.bitcast(_MAGIC - lax.shift_right_arithmetic(i, 1), jnp.float32)
    for _ in range(3):
        y = y * (1.5 - 0.5 * d * y * y)
    return y


NB = 4
ROWB = EC * H * 4
DEGQ = 16


def _sc_layer_body(compute_dinv, *refs):
    if compute_dinv:
        (src_hbm, dst_hbm, ew_hbm, feat_hbm, z_nodes, z_deg,
         part_out, dinv_out,
         src_blk, dst_blk, ew_blk, dinv_v, dtmp,
         r0, r1, r2, r3, cvals,
         acc_sh, feat_sh, deg_sh,
         g0, g1, g2, g3, s0, s1, s2, s3, dsem) = refs
    else:
        (src_hbm, dst_hbm, ew_hbm, feat_hbm, z_nodes, dinv_hbm,
         part_out,
         src_blk, dst_blk, ew_blk, dinv_v,
         r0, r1, r2, r3, cvals,
         acc_sh, feat_sh,
         g0, g1, g2, g3, s0, s1, s2, s3) = refs
    rows = (r0, r1, r2, r3)
    gsem = (g0, g1, g2, g3)
    ssem = (s0, s1, s2, s3)

    cid = lax.axis_index("c")
    sid = lax.axis_index("s")
    wid = cid * 16 + sid

    @pl.when(sid == 0)
    def _():
        pltpu.sync_copy(z_nodes, acc_sh)
        if compute_dinv:
            pltpu.sync_copy(z_deg, deg_sh)

    @pl.when(sid == 1)
    def _():
        pltpu.sync_copy(feat_hbm, feat_sh)

    plsc.subcore_barrier()

    if compute_dinv:
        pltpu.sync_copy(dst_hbm.at[pl.ds(sid * RTD, RTD)], dst_blk)
        pltpu.sync_copy(ew_hbm.at[pl.ds(sid * RTD, RTD)], ew_blk)

        def deg_step(j, _):
            pltpu.async_copy(ew_blk.at[j], deg_sh.at[dst_blk.at[j]], dsem,
                             add=True)

            @pl.when(j >= DEGQ)
            def _():
                pltpu.make_async_copy(
                    ew_blk.at[j], deg_sh.at[dst_blk.at[j]], dsem).wait()

            return 0

        lax.fori_loop(0, RTD, deg_step, 0)
        for _ in range(DEGQ):
            pltpu.make_async_copy(
                ew_blk.at[0], deg_sh.at[dst_blk.at[0]], dsem).wait()
        plsc.subcore_barrier()

        pltpu.sync_copy(deg_sh.at[pl.ds(sid * 640, 640)], dtmp)

        def inv_step(k, _):
            dtmp[pl.ds(k * 16, 16)] = _invsqrt16(dtmp[pl.ds(k * 16, 16)])
            return 0

        lax.fori_loop(0, 40, inv_step, 0)
        pltpu.sync_copy(dtmp, deg_sh.at[pl.ds(sid * 640, 640)])
        plsc.subcore_barrier()

        @pl.when(jnp.logical_and(cid == 0, sid == 0))
        def _():
            pltpu.sync_copy(deg_sh, dinv_out)

        pltpu.sync_copy(deg_sh, dinv_v)
    else:
        pltpu.sync_copy(dinv_hbm, dinv_v)

    pltpu.sync_copy(src_hbm.at[pl.ds(wid * RT, RT)], src_blk)
    if compute_dinv:
        pltpu.sync_copy(dst_hbm.at[pl.ds(wid * RT, RT)], dst_blk.at[pl.ds(0, RT)])
        pltpu.sync_copy(ew_hbm.at[pl.ds(wid * RT, RT)], ew_blk.at[pl.ds(0, RT)])
    else:
        pltpu.sync_copy(dst_hbm.at[pl.ds(wid * RT, RT)], dst_blk)
        pltpu.sync_copy(ew_hbm.at[pl.ds(wid * RT, RT)], ew_blk)

    pltpu.async_copy(feat_sh.at[src_blk.at[0]], rows[0], gsem[0])

    def msg_outer(jo, _):
        for b in range(NB):
            j = jo + b
            nb = (b + 1) % NB

            @pl.when(j + 1 < RT)
            def _():
                @pl.when(j + 1 >= NB)
                def _():
                    pltpu.make_async_copy(
                        rows[nb], acc_sh.at[dst_blk.at[j]], ssem[nb]).wait()

                pltpu.async_copy(feat_sh.at[src_blk.at[j + 1]], rows[nb],
                                 gsem[nb])

            for g in range(EC // 16):
                sl = pl.ds(g * 16, 16)
                c = (ew_blk[j, sl]
                     * plsc.load_gather(dinv_v, [src_blk[j, sl]])
                     * plsc.load_gather(dinv_v, [dst_blk[j, sl]]))
                cvals[sl] = c

            pltpu.make_async_copy(
                feat_sh.at[src_blk.at[j]], rows[b], gsem[b]).wait()

            def scale_step(e, _):
                c_b = plsc.load_gather(cvals, [jnp.full((16,), e, jnp.int32)])
                rows[b][e, :] = rows[b][e, :] * c_b
                return 0

            lax.fori_loop(0, EC, scale_step, 0, unroll=16)
            pltpu.async_copy(rows[b], acc_sh.at[dst_blk.at[j]], ssem[b],
                             add=True)
        return 0

    lax.fori_loop(0, RT // NB, lambda i, c: msg_outer(i * NB, c), 0)
    for b in range(NB):
        pltpu.make_async_copy(
            rows[b], acc_sh.at[dst_blk.at[0]], ssem[b]).wait()
    plsc.subcore_barrier()

    @pl.when(sid == 0)
    def _():
        pltpu.sync_copy(acc_sh, part_out.at[cid])


@functools.cache
def _make_sc_layer(compute_dinv):
    out_type = [jax.ShapeDtypeStruct((2, N, H), jnp.float32)]
    if compute_dinv:
        out_type.append(jax.ShapeDtypeStruct((NP,), jnp.float32))
    scratch = [
        pltpu.VMEM((RT, EC), jnp.int32),
        pltpu.VMEM((RTD if compute_dinv else RT, EC), jnp.int32),
        pltpu.VMEM((RTD if compute_dinv else RT, EC), jnp.float32),
        pltpu.VMEM((NP,), jnp.float32),
    ]
    if compute_dinv:
        scratch.append(pltpu.VMEM((640,), jnp.float32))
    scratch += [pltpu.VMEM((EC, H), jnp.float32) for _ in range(NB)]
    scratch += [
        pltpu.VMEM((EC,), jnp.float32),
        pltpu.VMEM_SHARED((N, H), jnp.float32),
        pltpu.VMEM_SHARED((N, H), jnp.float32),
    ]
    if compute_dinv:
        scratch.append(pltpu.VMEM_SHARED((NP,), jnp.float32))
    scratch += [pltpu.SemaphoreType.DMA] * (2 * NB)
    if compute_dinv:
        scratch.append(pltpu.SemaphoreType.DMA)

    return pl.kernel(
        functools.partial(_sc_layer_body, compute_dinv),
        out_type=tuple(out_type),
        mesh=plsc.VectorSubcoreMesh(core_axis_name="c", subcore_axis_name="s"),
        scratch_types=tuple(scratch),
        compiler_params=pltpu.CompilerParams(
            needs_layout_passes=False, use_tc_tiling_on_sc=False),
        name="gcn_sc_layer1" if compute_dinv else "gcn_sc_layer2",
    )


def _mm1_body(x_ref, w_ref, o_ref):
    o_ref[...] = jnp.dot(x_ref[...], w_ref[...],
                         preferred_element_type=jnp.float32)


def _mm2_body(part_ref, b_ref, w_ref, o_ref):
    h = jnp.maximum(part_ref[0] + part_ref[1] + b_ref[...], 0.0)
    o_ref[...] = jnp.dot(h, w_ref[...], preferred_element_type=jnp.float32)


def _pool_body(part_ref, b_ref, batch_ref, wl_ref, bl_ref, o_ref):
    h = jnp.maximum(part_ref[0] + part_ref[1] + b_ref[...], 0.0)
    iot = lax.broadcasted_iota(jnp.int32, (G, N), 0)
    oh = (iot == batch_ref[...]).astype(jnp.float32)
    sums = jnp.dot(oh, h, preferred_element_type=jnp.float32)
    cnts = jnp.sum(oh, axis=1, keepdims=True)
    g = sums / jnp.maximum(cnts, 1.0)
    o_ref[...] = jnp.dot(g, wl_ref[...],
                         preferred_element_type=jnp.float32) + bl_ref[...]


def kernel(x, edge_index, edge_attr, batch, W1, b1, W2, b2, Wl, bl):
    E = edge_attr.shape[0]
    pad = ER * EC - E - N
    loop_idx = jnp.arange(N, dtype=jnp.int32)
    zpad_i = jnp.zeros((pad,), jnp.int32)
    src_f = jnp.concatenate([edge_index[0], loop_idx, zpad_i]).reshape(ER, EC)
    dst_f = jnp.concatenate([edge_index[1], loop_idx, zpad_i]).reshape(ER, EC)
    ew_f = jnp.concatenate(
        [edge_attr, jnp.ones((N,), jnp.float32), jnp.zeros((pad,), jnp.float32)]
    ).reshape(ER, EC)

    z_nodes = jnp.zeros((N, H), jnp.float32)
    z_deg = jnp.zeros((NP,), jnp.float32)

    W2p = jnp.pad(W2, ((0, 0), (0, H - W2.shape[1])))
    b2p = jnp.pad(b2, (0, H - b2.shape[0])).reshape(1, H)
    Wlp = jnp.pad(Wl, ((0, H - Wl.shape[0]), (0, 0)))
    b1r = b1.reshape(1, H)

    xw1 = pl.pallas_call(
        _mm1_body,
        grid=(10,),
        in_specs=[
            pl.BlockSpec((N // 10, D_IN), lambda i: (i, 0)),
            pl.BlockSpec((D_IN, H), lambda i: (0, 0)),
        ],
        out_specs=pl.BlockSpec((N // 10, H), lambda i: (i, 0)),
        out_shape=jax.ShapeDtypeStruct((N, H), jnp.float32),
        name="gcn_mm1",
    )(x, W1)

    part1, dinv = _make_sc_layer(True)(src_f, dst_f, ew_f, xw1, z_nodes, z_deg)

    xw2 = pl.pallas_call(
        _mm2_body,
        out_shape=jax.ShapeDtypeStruct((N, H), jnp.float32),
        name="gcn_mm2",
    )(part1, b1r, W2p)

    (part2,) = _make_sc_layer(False)(src_f, dst_f, ew_f, xw2, z_nodes, dinv)

    out = pl.pallas_call(
        _pool_body,
        out_shape=jax.ShapeDtypeStruct((G, 1), jnp.float32),
        name="gcn_pool",
    )(part2, b2p, batch.reshape(1, N), Wlp, bl.reshape(1, 1))

    return out.reshape(-1)


# speedup vs baseline: 1.1102x; 1.0111x over previous
import functools

import jax
import jax.numpy as jnp
from jax import lax
from jax.experimental import pallas as pl
from jax.experimental.pallas import tpu as pltpu
from jax.experimental.pallas import tpu_sc as plsc

N = 10000
NP = 10240
G = 64
D_IN = 200
H = 16

EC = 128
NTILES = 32
ER = 2816
RT = ER // NTILES
RTD = ER // 16

_MAGIC = 0x5F3759DF


def _invsqrt16(d):
    i = plsc.bitcast(d, jnp.int32)
    y = plsc.bitcast(_MAGIC - lax.shift_right_arithmetic(i, 1), jnp.float32)
    for _ in range(3):
        y = y * (1.5 - 0.5 * d * y * y)
    return y


NB = 2
ROWB = EC * H * 4
DEGQ = 16


def _sc_layer_body(compute_dinv, *refs):
    it = iter(refs)
    src_hbm, dst_hbm, ew_hbm, feat_hbm, z_nodes = (next(it) for _ in range(5))
    if compute_dinv:
        z_deg, part_out, dinv_out = next(it), next(it), next(it)
    else:
        dinv_hbm, part_out = next(it), next(it)
    src_blk, dst_blk, ew_blk, dinv_v = (next(it) for _ in range(4))
    if compute_dinv:
        dtmp = next(it)
    rows = tuple(next(it) for _ in range(NB))
    cvals, acc_sh, feat_sh = next(it), next(it), next(it)
    if compute_dinv:
        deg_sh = next(it)
    gsem = tuple(next(it) for _ in range(NB))
    ssem = tuple(next(it) for _ in range(NB))
    if compute_dinv:
        dsem = next(it)

    cid = lax.axis_index("c")
    sid = lax.axis_index("s")
    wid = cid * 16 + sid

    @pl.when(sid == 0)
    def _():
        pltpu.sync_copy(z_nodes, acc_sh)
        if compute_dinv:
            pltpu.sync_copy(z_deg, deg_sh)

    @pl.when(sid == 1)
    def _():
        pltpu.sync_copy(feat_hbm, feat_sh)

    plsc.subcore_barrier()

    if compute_dinv:
        pltpu.sync_copy(dst_hbm.at[pl.ds(sid * RTD, RTD)], dst_blk)
        pltpu.sync_copy(ew_hbm.at[pl.ds(sid * RTD, RTD)], ew_blk)

        def deg_step(j, _):
            pltpu.async_copy(ew_blk.at[j], deg_sh.at[dst_blk.at[j]], dsem,
                             add=True)

            @pl.when(j >= DEGQ)
            def _():
                pltpu.make_async_copy(
                    ew_blk.at[j], deg_sh.at[dst_blk.at[j]], dsem).wait()

            return 0

        lax.fori_loop(0, RTD, deg_step, 0)
        for _ in range(DEGQ):
            pltpu.make_async_copy(
                ew_blk.at[0], deg_sh.at[dst_blk.at[0]], dsem).wait()
        plsc.subcore_barrier()

        pltpu.sync_copy(deg_sh.at[pl.ds(sid * 640, 640)], dtmp)

        def inv_step(k, _):
            dtmp[pl.ds(k * 16, 16)] = _invsqrt16(dtmp[pl.ds(k * 16, 16)])
            return 0

        lax.fori_loop(0, 40, inv_step, 0)
        pltpu.sync_copy(dtmp, deg_sh.at[pl.ds(sid * 640, 640)])
        plsc.subcore_barrier()

        @pl.when(jnp.logical_and(cid == 0, sid == 0))
        def _():
            pltpu.sync_copy(deg_sh, dinv_out)

        pltpu.sync_copy(deg_sh, dinv_v)
    else:
        pltpu.sync_copy(dinv_hbm, dinv_v)

    pltpu.sync_copy(src_hbm.at[pl.ds(wid * RT, RT)], src_blk)
    if compute_dinv:
        pltpu.sync_copy(dst_hbm.at[pl.ds(wid * RT, RT)], dst_blk.at[pl.ds(0, RT)])
        pltpu.sync_copy(ew_hbm.at[pl.ds(wid * RT, RT)], ew_blk.at[pl.ds(0, RT)])
    else:
        pltpu.sync_copy(dst_hbm.at[pl.ds(wid * RT, RT)], dst_blk)
        pltpu.sync_copy(ew_hbm.at[pl.ds(wid * RT, RT)], ew_blk)

    pltpu.async_copy(feat_sh.at[src_blk.at[0]], rows[0], gsem[0])

    def msg_outer(jo, _):
        for b in range(NB):
            j = jo + b
            nb = (b + 1) % NB

            @pl.when(j + 1 < RT)
            def _():
                @pl.when(j + 1 >= NB)
                def _():
                    pltpu.make_async_copy(
                        rows[nb], acc_sh.at[dst_blk.at[j]], ssem[nb]).wait()

                pltpu.async_copy(feat_sh.at[src_blk.at[j + 1]], rows[nb],
                                 gsem[nb])

            for g in range(EC // 16):
                sl = pl.ds(g * 16, 16)
                c = (ew_blk[j, sl]
                     * plsc.load_gather(dinv_v, [src_blk[j, sl]])
                     * plsc.load_gather(dinv_v, [dst_blk[j, sl]]))
                cvals[sl] = c

            pltpu.make_async_copy(
                feat_sh.at[src_blk.at[j]], rows[b], gsem[b]).wait()

            def scale_step(e, _):
                c_b = plsc.load_gather(cvals, [jnp.full((16,), e, jnp.int32)])
                rows[b][e, :] = rows[b][e, :] * c_b
                return 0

            lax.fori_loop(0, EC, scale_step, 0, unroll=8)
            pltpu.async_copy(rows[b], acc_sh.at[dst_blk.at[j]], ssem[b],
                             add=True)
        return 0

    lax.fori_loop(0, RT // NB, lambda i, c: msg_outer(i * NB, c), 0)
    for b in range(NB):
        pltpu.make_async_copy(
            rows[b], acc_sh.at[dst_blk.at[0]], ssem[b]).wait()
    plsc.subcore_barrier()

    @pl.when(sid == 0)
    def _():
        pltpu.sync_copy(acc_sh, part_out.at[cid])


@functools.cache
def _make_sc_layer(compute_dinv):
    out_type = [jax.ShapeDtypeStruct((2, N, H), jnp.float32)]
    if compute_dinv:
        out_type.append(jax.ShapeDtypeStruct((NP,), jnp.float32))
    scratch = [
        pltpu.VMEM((RT, EC), jnp.int32),
        pltpu.VMEM((RTD if compute_dinv else RT, EC), jnp.int32),
        pltpu.VMEM((RTD if compute_dinv else RT, EC), jnp.float32),
        pltpu.VMEM((NP,), jnp.float32),
    ]
    if compute_dinv:
        scratch.append(pltpu.VMEM((640,), jnp.float32))
    scratch += [pltpu.VMEM((EC, H), jnp.float32) for _ in range(NB)]
    scratch += [
        pltpu.VMEM((EC,), jnp.float32),
        pltpu.VMEM_SHARED((N, H), jnp.float32),
        pltpu.VMEM_SHARED((N, H), jnp.float32),
    ]
    if compute_dinv:
        scratch.append(pltpu.VMEM_SHARED((NP,), jnp.float32))
    scratch += [pltpu.SemaphoreType.DMA] * (2 * NB)
    if compute_dinv:
        scratch.append(pltpu.SemaphoreType.DMA)

    return pl.kernel(
        functools.partial(_sc_layer_body, compute_dinv),
        out_type=tuple(out_type),
        mesh=plsc.VectorSubcoreMesh(core_axis_name="c", subcore_axis_name="s"),
        scratch_types=tuple(scratch),
        compiler_params=pltpu.CompilerParams(
            needs_layout_passes=False, use_tc_tiling_on_sc=False),
        name="gcn_sc_layer1" if compute_dinv else "gcn_sc_layer2",
    )


def _mm1_body(x_ref, w_ref, o_ref):
    o_ref[...] = jnp.dot(x_ref[...], w_ref[...],
                         preferred_element_type=jnp.float32)


def _mm2_body(part_ref, b_ref, w_ref, o_ref):
    h = jnp.maximum(part_ref[0] + part_ref[1] + b_ref[...], 0.0)
    o_ref[...] = jnp.dot(h, w_ref[...], preferred_element_type=jnp.float32)


def _pool_body(part_ref, b_ref, batch_ref, wl_ref, bl_ref, o_ref):
    h = jnp.maximum(part_ref[0] + part_ref[1] + b_ref[...], 0.0)
    iot = lax.broadcasted_iota(jnp.int32, (G, N), 0)
    oh = (iot == batch_ref[...]).astype(jnp.float32)
    sums = jnp.dot(oh, h, preferred_element_type=jnp.float32)
    cnts = jnp.sum(oh, axis=1, keepdims=True)
    g = sums / jnp.maximum(cnts, 1.0)
    o_ref[...] = jnp.dot(g, wl_ref[...],
                         preferred_element_type=jnp.float32) + bl_ref[...]


def kernel(x, edge_index, edge_attr, batch, W1, b1, W2, b2, Wl, bl):
    E = edge_attr.shape[0]
    pad = ER * EC - E - N
    loop_idx = jnp.arange(N, dtype=jnp.int32)
    zpad_i = jnp.zeros((pad,), jnp.int32)
    src_f = jnp.concatenate([edge_index[0], loop_idx, zpad_i]).reshape(ER, EC)
    dst_f = jnp.concatenate([edge_index[1], loop_idx, zpad_i]).reshape(ER, EC)
    ew_f = jnp.concatenate(
        [edge_attr, jnp.ones((N,), jnp.float32), jnp.zeros((pad,), jnp.float32)]
    ).reshape(ER, EC)

    z_nodes = jnp.zeros((N, H), jnp.float32)
    z_deg = jnp.zeros((NP,), jnp.float32)

    W2p = jnp.pad(W2, ((0, 0), (0, H - W2.shape[1])))
    b2p = jnp.pad(b2, (0, H - b2.shape[0])).reshape(1, H)
    Wlp = jnp.pad(Wl, ((0, H - Wl.shape[0]), (0, 0)))
    b1r = b1.reshape(1, H)

    xw1 = pl.pallas_call(
        _mm1_body,
        grid=(10,),
        in_specs=[
            pl.BlockSpec((N // 10, D_IN), lambda i: (i, 0)),
            pl.BlockSpec((D_IN, H), lambda i: (0, 0)),
        ],
        out_specs=pl.BlockSpec((N // 10, H), lambda i: (i, 0)),
        out_shape=jax.ShapeDtypeStruct((N, H), jnp.float32),
        name="gcn_mm1",
    )(x, W1)

    part1, dinv = _make_sc_layer(True)(src_f, dst_f, ew_f, xw1, z_nodes, z_deg)

    xw2 = pl.pallas_call(
        _mm2_body,
        out_shape=jax.ShapeDtypeStruct((N, H), jnp.float32),
        name="gcn_mm2",
    )(part1, b1r, W2p)

    (part2,) = _make_sc_layer(False)(src_f, dst_f, ew_f, xw2, z_nodes, dinv)

    out = pl.pallas_call(
        _pool_body,
        out_shape=jax.ShapeDtypeStruct((G, 1), jnp.float32),
        name="gcn_pool",
    )(part2, b2p, batch.reshape(1, N), Wlp, bl.reshape(1, 1))

    return out.reshape(-1)


# speedup vs baseline: 1.4368x; 1.2942x over previous
import functools

import jax
import jax.numpy as jnp
from jax import lax
from jax.experimental import pallas as pl
from jax.experimental.pallas import tpu as pltpu
from jax.experimental.pallas import tpu_sc as plsc

N = 10000
NP = 10240
G = 64
D_IN = 200
H = 16

EC = 128
NTILES = 32
ER = 2816
RT = ER // NTILES
RTD = ER // 16

_MAGIC = 0x5F3759DF


def _invsqrt16(d):
    i = plsc.bitcast(d, jnp.int32)
    y = plsc.bitcast(_MAGIC - lax.shift_right_arithmetic(i, 1), jnp.float32)
    for _ in range(3):
        y = y * (1.5 - 0.5 * d * y * y)
    return y


NB = 4
ROWB = EC * H * 4
DEGQ = 16


def _sc_layer_body(compute_dinv, *refs):
    it = iter(refs)
    src_hbm, dst_hbm, ew_hbm, feat_hbm, z_nodes = (next(it) for _ in range(5))
    if compute_dinv:
        z_deg, part_out, dinv_out = next(it), next(it), next(it)
    else:
        dinv_hbm, part_out = next(it), next(it)
    src_blk, dst_blk, ew_blk, dinv_v = (next(it) for _ in range(4))
    if compute_dinv:
        dtmp = next(it)
    rows = tuple(next(it) for _ in range(NB))
    cvals, acc_sh, feat_sh = next(it), next(it), next(it)
    if compute_dinv:
        deg_sh = next(it)
    gsem = tuple(next(it) for _ in range(NB))
    ssem = tuple(next(it) for _ in range(NB))
    if compute_dinv:
        dsem = next(it)

    cid = lax.axis_index("c")
    sid = lax.axis_index("s")
    wid = cid * 16 + sid

    @pl.when(sid == 0)
    def _():
        pltpu.sync_copy(z_nodes, acc_sh)
        if compute_dinv:
            pltpu.sync_copy(z_deg, deg_sh)

    @pl.when(sid == 1)
    def _():
        pltpu.sync_copy(feat_hbm, feat_sh)

    plsc.subcore_barrier()

    if compute_dinv:
        pltpu.sync_copy(dst_hbm.at[pl.ds(sid * RTD, RTD)], dst_blk)
        pltpu.sync_copy(ew_hbm.at[pl.ds(sid * RTD, RTD)], ew_blk)

        def deg_step(j, _):
            pltpu.async_copy(ew_blk.at[j], deg_sh.at[dst_blk.at[j]], dsem,
                             add=True)

            @pl.when(j >= DEGQ)
            def _():
                pltpu.make_async_copy(
                    ew_blk.at[j], deg_sh.at[dst_blk.at[j]], dsem).wait()

            return 0

        lax.fori_loop(0, RTD, deg_step, 0)
        for _ in range(DEGQ):
            pltpu.make_async_copy(
                ew_blk.at[0], deg_sh.at[dst_blk.at[0]], dsem).wait()
        plsc.subcore_barrier()

        pltpu.sync_copy(deg_sh.at[pl.ds(sid * 640, 640)], dtmp)

        def inv_step(k, _):
            dtmp[pl.ds(k * 16, 16)] = _invsqrt16(dtmp[pl.ds(k * 16, 16)])
            return 0

        lax.fori_loop(0, 40, inv_step, 0)
        pltpu.sync_copy(dtmp, deg_sh.at[pl.ds(sid * 640, 640)])
        plsc.subcore_barrier()

        @pl.when(jnp.logical_and(cid == 0, sid == 0))
        def _():
            pltpu.sync_copy(deg_sh, dinv_out)

        pltpu.sync_copy(deg_sh, dinv_v)
    else:
        pltpu.sync_copy(dinv_hbm, dinv_v)

    pltpu.sync_copy(src_hbm.at[pl.ds(wid * RT, RT)], src_blk)
    if compute_dinv:
        pltpu.sync_copy(dst_hbm.at[pl.ds(wid * RT, RT)], dst_blk.at[pl.ds(0, RT)])
        pltpu.sync_copy(ew_hbm.at[pl.ds(wid * RT, RT)], ew_blk.at[pl.ds(0, RT)])
    else:
        pltpu.sync_copy(dst_hbm.at[pl.ds(wid * RT, RT)], dst_blk)
        pltpu.sync_copy(ew_hbm.at[pl.ds(wid * RT, RT)], ew_blk)

    pltpu.async_copy(feat_sh.at[src_blk.at[0]], rows[0], gsem[0])

    def msg_outer(jo, _):
        for b in range(NB):
            j = jo + b
            nb = (b + 1) % NB

            @pl.when(j + 1 < RT)
            def _():
                @pl.when(j + 1 >= NB)
                def _():
                    pltpu.make_async_copy(
                        rows[nb], acc_sh.at[dst_blk.at[j]], ssem[nb]).wait()

                pltpu.async_copy(feat_sh.at[src_blk.at[j + 1]], rows[nb],
                                 gsem[nb])

            for g in range(EC // 16):
                sl = pl.ds(g * 16, 16)
                c = (ew_blk[j, sl]
                     * plsc.load_gather(dinv_v, [src_blk[j, sl]])
                     * plsc.load_gather(dinv_v, [dst_blk[j, sl]]))
                cvals[sl] = c

            pltpu.make_async_copy(
                feat_sh.at[src_blk.at[j]], rows[b], gsem[b]).wait()

            dnums = lax.GatherDimensionNumbers(
                offset_dims=(), collapsed_slice_dims=(0,),
                start_index_map=(0,))

            def scale_grp(g, _):
                c = cvals[pl.ds(g * 16, 16)]
                for i in range(16):
                    c_b = lax.gather(
                        c, jnp.full((16, 1), i, jnp.int32), dnums, (1,),
                        mode=lax.GatherScatterMode.PROMISE_IN_BOUNDS)
                    e = g * 16 + i
                    rows[b][e, :] = rows[b][e, :] * c_b
                return 0

            lax.fori_loop(0, EC // 16, scale_grp, 0)
            pltpu.async_copy(rows[b], acc_sh.at[dst_blk.at[j]], ssem[b],
                             add=True)
        return 0

    lax.fori_loop(0, RT // NB, lambda i, c: msg_outer(i * NB, c), 0)
    for b in range(NB):
        pltpu.make_async_copy(
            rows[b], acc_sh.at[dst_blk.at[0]], ssem[b]).wait()
    plsc.subcore_barrier()

    @pl.when(sid == 0)
    def _():
        pltpu.sync_copy(acc_sh, part_out.at[cid])


@functools.cache
def _make_sc_layer(compute_dinv):
    out_type = [jax.ShapeDtypeStruct((2, N, H), jnp.float32)]
    if compute_dinv:
        out_type.append(jax.ShapeDtypeStruct((NP,), jnp.float32))
    scratch = [
        pltpu.VMEM((RT, EC), jnp.int32),
        pltpu.VMEM((RTD if compute_dinv else RT, EC), jnp.int32),
        pltpu.VMEM((RTD if compute_dinv else RT, EC), jnp.float32),
        pltpu.VMEM((NP,), jnp.float32),
    ]
    if compute_dinv:
        scratch.append(pltpu.VMEM((640,), jnp.float32))
    scratch += [pltpu.VMEM((EC, H), jnp.float32) for _ in range(NB)]
    scratch += [
        pltpu.VMEM((EC,), jnp.float32),
        pltpu.VMEM_SHARED((N, H), jnp.float32),
        pltpu.VMEM_SHARED((N, H), jnp.float32),
    ]
    if compute_dinv:
        scratch.append(pltpu.VMEM_SHARED((NP,), jnp.float32))
    scratch += [pltpu.SemaphoreType.DMA] * (2 * NB)
    if compute_dinv:
        scratch.append(pltpu.SemaphoreType.DMA)

    return pl.kernel(
        functools.partial(_sc_layer_body, compute_dinv),
        out_type=tuple(out_type),
        mesh=plsc.VectorSubcoreMesh(core_axis_name="c", subcore_axis_name="s"),
        scratch_types=tuple(scratch),
        compiler_params=pltpu.CompilerParams(
            needs_layout_passes=False, use_tc_tiling_on_sc=False),
        name="gcn_sc_layer1" if compute_dinv else "gcn_sc_layer2",
    )


def _mm1_body(x_ref, w_ref, o_ref):
    o_ref[...] = jnp.dot(x_ref[...], w_ref[...],
                         preferred_element_type=jnp.float32)


def _mm2_body(part_ref, b_ref, w_ref, o_ref):
    h = jnp.maximum(part_ref[0] + part_ref[1] + b_ref[...], 0.0)
    o_ref[...] = jnp.dot(h, w_ref[...], preferred_element_type=jnp.float32)


def _pool_body(part_ref, b_ref, batch_ref, wl_ref, bl_ref, o_ref):
    h = jnp.maximum(part_ref[0] + part_ref[1] + b_ref[...], 0.0)
    iot = lax.broadcasted_iota(jnp.int32, (G, N), 0)
    oh = (iot == batch_ref[...]).astype(jnp.float32)
    sums = jnp.dot(oh, h, preferred_element_type=jnp.float32)
    cnts = jnp.sum(oh, axis=1, keepdims=True)
    g = sums / jnp.maximum(cnts, 1.0)
    o_ref[...] = jnp.dot(g, wl_ref[...],
                         preferred_element_type=jnp.float32) + bl_ref[...]


def kernel(x, edge_index, edge_attr, batch, W1, b1, W2, b2, Wl, bl):
    E = edge_attr.shape[0]
    pad = ER * EC - E - N
    loop_idx = jnp.arange(N, dtype=jnp.int32)
    zpad_i = jnp.zeros((pad,), jnp.int32)
    src_f = jnp.concatenate([edge_index[0], loop_idx, zpad_i]).reshape(ER, EC)
    dst_f = jnp.concatenate([edge_index[1], loop_idx, zpad_i]).reshape(ER, EC)
    ew_f = jnp.concatenate(
        [edge_attr, jnp.ones((N,), jnp.float32), jnp.zeros((pad,), jnp.float32)]
    ).reshape(ER, EC)

    z_nodes = jnp.zeros((N, H), jnp.float32)
    z_deg = jnp.zeros((NP,), jnp.float32)

    W2p = jnp.pad(W2, ((0, 0), (0, H - W2.shape[1])))
    b2p = jnp.pad(b2, (0, H - b2.shape[0])).reshape(1, H)
    Wlp = jnp.pad(Wl, ((0, H - Wl.shape[0]), (0, 0)))
    b1r = b1.reshape(1, H)

    xw1 = pl.pallas_call(
        _mm1_body,
        grid=(10,),
        in_specs=[
            pl.BlockSpec((N // 10, D_IN), lambda i: (i, 0)),
            pl.BlockSpec((D_IN, H), lambda i: (0, 0)),
        ],
        out_specs=pl.BlockSpec((N // 10, H), lambda i: (i, 0)),
        out_shape=jax.ShapeDtypeStruct((N, H), jnp.float32),
        name="gcn_mm1",
    )(x, W1)

    part1, dinv = _make_sc_layer(True)(src_f, dst_f, ew_f, xw1, z_nodes, z_deg)

    xw2 = pl.pallas_call(
        _mm2_body,
        out_shape=jax.ShapeDtypeStruct((N, H), jnp.float32),
        name="gcn_mm2",
    )(part1, b1r, W2p)

    (part2,) = _make_sc_layer(False)(src_f, dst_f, ew_f, xw2, z_nodes, dinv)

    out = pl.pallas_call(
        _pool_body,
        out_shape=jax.ShapeDtypeStruct((G, 1), jnp.float32),
        name="gcn_pool",
    )(part2, b2p, batch.reshape(1, N), Wlp, bl.reshape(1, 1))

    return out.reshape(-1)


# speedup vs baseline: 2.1802x; 1.5174x over previous
import functools

import jax
import jax.numpy as jnp
from jax import lax
from jax.experimental import pallas as pl
from jax.experimental.pallas import tpu as pltpu
from jax.experimental.pallas import tpu_sc as plsc

N = 10000
NP = 10240
G = 64
D_IN = 200
H = 16

EC = 128
NTILES = 32
ER = 2500
MROWS = 79
DROWS = 157

NB = 4
ROWB = EC * H * 4
DEGQ = 16

_MAGIC = 0x5F3759DF


def _invsqrt16(d):
    i = plsc.bitcast(d, jnp.int32)
    y = plsc.bitcast(_MAGIC - lax.shift_right_arithmetic(i, 1), jnp.float32)
    for _ in range(3):
        y = y * (1.5 - 0.5 * d * y * y)
    return y


def _sc_layer_body(compute_dinv, *refs):
    it = iter(refs)
    src_hbm, dst_hbm, ew_hbm, feat_hbm, init_hbm = (next(it) for _ in range(5))
    if compute_dinv:
        z_deg, part_out, dinv_out = next(it), next(it), next(it)
    else:
        dinv_hbm, part_out = next(it), next(it)
    src_blk, dst_blk, ew_blk, dinv_v = (next(it) for _ in range(4))
    if compute_dinv:
        dtmp = next(it)
    rows = tuple(next(it) for _ in range(NB))
    cvals, acc_sh, feat_sh = next(it), next(it), next(it)
    if compute_dinv:
        deg_sh = next(it)
    gsem = tuple(next(it) for _ in range(NB))
    ssem = tuple(next(it) for _ in range(NB))
    if compute_dinv:
        dsem = next(it)

    cid = lax.axis_index("c")
    sid = lax.axis_index("s")
    wid = cid * 16 + sid

    mlo = wid * ER // NTILES
    mcnt = (wid + 1) * ER // NTILES - mlo

    @pl.when(sid == 0)
    def _():
        pltpu.sync_copy(init_hbm, acc_sh)
        if compute_dinv:
            pltpu.sync_copy(z_deg, deg_sh)

    @pl.when(sid == 1)
    def _():
        pltpu.sync_copy(feat_hbm, feat_sh)

    plsc.subcore_barrier()

    if compute_dinv:
        dlo = sid * ER // 16
        dcnt = (sid + 1) * ER // 16 - dlo
        pltpu.sync_copy(dst_hbm.at[pl.ds(dlo, DROWS)], dst_blk)
        pltpu.sync_copy(ew_hbm.at[pl.ds(dlo, DROWS)], ew_blk)

        def deg_step(j, _):
            pltpu.async_copy(ew_blk.at[j], deg_sh.at[dst_blk.at[j]], dsem,
                             add=True)

            @pl.when(j >= DEGQ)
            def _():
                pltpu.make_async_copy(
                    ew_blk.at[j], deg_sh.at[dst_blk.at[j]], dsem).wait()

            return 0

        lax.fori_loop(0, dcnt, deg_step, 0)
        for _ in range(DEGQ):
            pltpu.make_async_copy(
                ew_blk.at[0], deg_sh.at[dst_blk.at[0]], dsem).wait()
        plsc.subcore_barrier()

        pltpu.sync_copy(deg_sh.at[pl.ds(sid * 640, 640)], dtmp)

        def inv_step(k, _):
            sl = pl.ds(k * 16, 16)
            dtmp[sl] = _invsqrt16(dtmp[sl] + 1.0)
            return 0

        lax.fori_loop(0, 40, inv_step, 0)
        pltpu.sync_copy(dtmp, deg_sh.at[pl.ds(sid * 640, 640)])
        plsc.subcore_barrier()

        @pl.when(jnp.logical_and(cid == 0, sid == 0))
        def _():
            pltpu.sync_copy(deg_sh, dinv_out)

        pltpu.sync_copy(deg_sh, dinv_v)
    else:
        pltpu.sync_copy(dinv_hbm, dinv_v)

    pltpu.sync_copy(src_hbm.at[pl.ds(mlo, MROWS)], src_blk)
    pltpu.sync_copy(dst_hbm.at[pl.ds(mlo, MROWS)],
                    dst_blk.at[pl.ds(0, MROWS)])
    pltpu.sync_copy(ew_hbm.at[pl.ds(mlo, MROWS)],
                    ew_blk.at[pl.ds(0, MROWS)])

    pltpu.async_copy(feat_sh.at[src_blk.at[0]], rows[0], gsem[0])

    dnums = lax.GatherDimensionNumbers(
        offset_dims=(), collapsed_slice_dims=(0,), start_index_map=(0,))

    def msg_outer(jo, _):
        for b in range(NB):
            j = jo * NB + b
            nb = (b + 1) % NB

            @pl.when(j < mcnt)
            def _():
                @pl.when(j + 1 < mcnt)
                def _():
                    @pl.when(j + 1 >= NB)
                    def _():
                        pltpu.make_async_copy(
                            rows[nb], acc_sh.at[dst_blk.at[j]],
                            ssem[nb]).wait()

                    pltpu.async_copy(feat_sh.at[src_blk.at[j + 1]], rows[nb],
                                     gsem[nb])

                for g in range(EC // 16):
                    sl = pl.ds(g * 16, 16)
                    c = (ew_blk[j, sl]
                         * plsc.load_gather(dinv_v, [src_blk[j, sl]])
                         * plsc.load_gather(dinv_v, [dst_blk[j, sl]]))
                    cvals[sl] = c

                pltpu.make_async_copy(
                    feat_sh.at[src_blk.at[j]], rows[b], gsem[b]).wait()

                def scale_grp(g, _):
                    c = cvals[pl.ds(g * 16, 16)]
                    for i in range(16):
                        c_b = lax.gather(
                            c, jnp.full((16, 1), i, jnp.int32), dnums, (1,),
                            mode=lax.GatherScatterMode.PROMISE_IN_BOUNDS)
                        e = g * 16 + i
                        rows[b][e, :] = rows[b][e, :] * c_b
                    return 0

                lax.fori_loop(0, EC // 16, scale_grp, 0)
                pltpu.async_copy(rows[b], acc_sh.at[dst_blk.at[j]], ssem[b],
                                 add=True)

        return 0

    lax.fori_loop(0, (mcnt + NB - 1) // NB, msg_outer, 0)
    for b in range(NB):
        pltpu.make_async_copy(
            rows[b], acc_sh.at[dst_blk.at[0]], ssem[b]).wait()
    plsc.subcore_barrier()

    @pl.when(sid == 0)
    def _():
        pltpu.sync_copy(acc_sh, part_out.at[cid])


@functools.cache
def _make_sc_layer(compute_dinv):
    out_type = [jax.ShapeDtypeStruct((2, N, H), jnp.float32)]
    if compute_dinv:
        out_type.append(jax.ShapeDtypeStruct((NP,), jnp.float32))
    scratch = [
        pltpu.VMEM((MROWS, EC), jnp.int32),
        pltpu.VMEM((DROWS if compute_dinv else MROWS, EC), jnp.int32),
        pltpu.VMEM((DROWS if compute_dinv else MROWS, EC), jnp.float32),
        pltpu.VMEM((NP,), jnp.float32),
    ]
    if compute_dinv:
        scratch.append(pltpu.VMEM((640,), jnp.float32))
    scratch += [pltpu.VMEM((EC, H), jnp.float32) for _ in range(NB)]
    scratch += [
        pltpu.VMEM((EC,), jnp.float32),
        pltpu.VMEM_SHARED((N, H), jnp.float32),
        pltpu.VMEM_SHARED((N, H), jnp.float32),
    ]
    if compute_dinv:
        scratch.append(pltpu.VMEM_SHARED((NP,), jnp.float32))
    scratch += [pltpu.SemaphoreType.DMA] * (2 * NB)
    if compute_dinv:
        scratch.append(pltpu.SemaphoreType.DMA)

    return pl.kernel(
        functools.partial(_sc_layer_body, compute_dinv),
        out_type=tuple(out_type),
        mesh=plsc.VectorSubcoreMesh(core_axis_name="c", subcore_axis_name="s"),
        scratch_types=tuple(scratch),
        compiler_params=pltpu.CompilerParams(
            needs_layout_passes=False, use_tc_tiling_on_sc=False),
        name="gcn_sc_layer1" if compute_dinv else "gcn_sc_layer2",
    )


def _mm1_body(x_ref, w_ref, o_ref):
    o_ref[...] = jnp.dot(x_ref[...], w_ref[...],
                         preferred_element_type=jnp.float32)


def _mm2_body(part_ref, b_ref, w_ref, xw1_ref, dinv_ref, xw2_ref, init2_ref):
    d2 = dinv_ref[...] * dinv_ref[...]
    h = jnp.maximum(part_ref[0] + part_ref[1] + d2 * xw1_ref[...]
                    + b_ref[...], 0.0)
    xw2 = jnp.dot(h, w_ref[...], preferred_element_type=jnp.float32)
    xw2_ref[...] = xw2
    init2_ref[...] = 0.5 * d2 * xw2


def _pool_body(part_ref, b_ref, batch_ref, wl_ref, bl_ref, o_ref):
    h = jnp.maximum(part_ref[0] + part_ref[1] + b_ref[...], 0.0)
    iot = lax.broadcasted_iota(jnp.int32, (G, N), 0)
    oh = (iot == batch_ref[...]).astype(jnp.float32)
    sums = jnp.dot(oh, h, preferred_element_type=jnp.float32)
    cnts = jnp.sum(oh, axis=1, keepdims=True)
    g = sums / jnp.maximum(cnts, 1.0)
    o_ref[...] = jnp.dot(g, wl_ref[...],
                         preferred_element_type=jnp.float32) + bl_ref[...]


def kernel(x, edge_index, edge_attr, batch, W1, b1, W2, b2, Wl, bl):
    src_f = edge_index[0].reshape(ER, EC)
    dst_f = edge_index[1].reshape(ER, EC)
    ew_f = edge_attr.reshape(ER, EC)

    z_nodes = jnp.zeros((N, H), jnp.float32)
    z_deg = jnp.zeros((NP,), jnp.float32)

    W2p = jnp.pad(W2, ((0, 0), (0, H - W2.shape[1])))
    b2p = jnp.pad(b2, (0, H - b2.shape[0])).reshape(1, H)
    Wlp = jnp.pad(Wl, ((0, H - Wl.shape[0]), (0, 0)))
    b1r = b1.reshape(1, H)

    xw1 = pl.pallas_call(
        _mm1_body,
        grid=(10,),
        in_specs=[
            pl.BlockSpec((N // 10, D_IN), lambda i: (i, 0)),
            pl.BlockSpec((D_IN, H), lambda i: (0, 0)),
        ],
        out_specs=pl.BlockSpec((N // 10, H), lambda i: (i, 0)),
        out_shape=jax.ShapeDtypeStruct((N, H), jnp.float32),
        name="gcn_mm1",
    )(x, W1)

    part1, dinv = _make_sc_layer(True)(src_f, dst_f, ew_f, xw1, z_nodes,
                                       z_deg)
    dinv2d = dinv[:N].reshape(N, 1)

    xw2, init2 = pl.pallas_call(
        _mm2_body,
        out_shape=(jax.ShapeDtypeStruct((N, H), jnp.float32),
                   jax.ShapeDtypeStruct((N, H), jnp.float32)),
        name="gcn_mm2",
    )(part1, b1r, W2p, xw1, dinv2d)

    (part2,) = _make_sc_layer(False)(src_f, dst_f, ew_f, xw2, init2, dinv)

    out = pl.pallas_call(
        _pool_body,
        out_shape=jax.ShapeDtypeStruct((G, 1), jnp.float32),
        name="gcn_pool",
    )(part2, b2p, batch.reshape(1, N), Wlp, bl.reshape(1, 1))

    return out.reshape(-1)
